# baseline (speedup 1.0000x reference)
import functools

import jax
import jax.numpy as jnp
from jax import lax
from jax.experimental import pallas as pl
from jax.experimental.pallas import tpu as pltpu
from jax.experimental.pallas import tpu_sc as plsc

_VOCAB = 100000
_EMBED = 128
_HIDDEN = 128
_CTX = 200

_NC = 2
_NS = 16
_NW = _NC * _NS
_IDX_PER_W = 8
_ACTIVE_W = _CTX // _IDX_PER_W

_VT = 10000
_NT = _VOCAB // _VT


def _sc_gather_sum(idx_hbm, table_hbm, out_hbm, idx_v, rows_v, sum_v, sem):
    wid = lax.axis_index("s") * _NC + lax.axis_index("c")
    zero = jnp.zeros((16,), jnp.float32)
    for c in range(_EMBED // 16):
        sum_v[0, pl.ds(c * 16, 16)] = zero

    @pl.when(wid < _ACTIVE_W)
    def _():
        base = wid * _IDX_PER_W
        pltpu.sync_copy(idx_hbm.at[pl.ds(base, _IDX_PER_W)], idx_v)
        pltpu.async_copy(table_hbm.at[idx_v], rows_v, sem).wait()
        for c in range(_EMBED // 16):
            acc = rows_v[0, pl.ds(c * 16, 16)]
            for r in range(1, _IDX_PER_W):
                acc = acc + rows_v[r, pl.ds(c * 16, 16)]
            sum_v[0, pl.ds(c * 16, 16)] = acc

    pltpu.sync_copy(sum_v, out_hbm.at[pl.ds(wid, 1)])


def _sc_call(idx, table):
    return pl.kernel(
        _sc_gather_sum,
        mesh=plsc.VectorSubcoreMesh(core_axis_name="c", subcore_axis_name="s"),
        out_type=jax.ShapeDtypeStruct((_NW, _EMBED), jnp.float32),
        scratch_types=[
            pltpu.VMEM((_IDX_PER_W,), jnp.int32),
            pltpu.VMEM((_IDX_PER_W, _EMBED), jnp.float32),
            pltpu.VMEM((1, _EMBED), jnp.float32),
            pltpu.SemaphoreType.DMA,
        ],
    )(idx, table)


def _tc_mlp(partials_ref, w1_ref, b1_ref, w2_ref, b2_ref, out_ref,
            h_ref, m_ref, s_ref):
    i = pl.program_id(0)

    @pl.when(i == 0)
    def _():
        emb = jnp.sum(partials_ref[...], axis=0, keepdims=True)
        h = lax.dot_general(emb, w1_ref[...], (((1,), (1,)), ((), ())),
                            preferred_element_type=jnp.float32)
        h_ref[...] = jnp.maximum(h + b1_ref[...], 0.0)
        m_ref[0] = -jnp.inf
        s_ref[0] = 0.0

    logits = lax.dot_general(h_ref[...], w2_ref[...], (((1,), (1,)), ((), ())),
                             preferred_element_type=jnp.float32) + b2_ref[0]
    out_ref[pl.ds(i, 1), :] = logits

    m_old = m_ref[0]
    m_new = jnp.maximum(m_old, jnp.max(logits))
    s_ref[0] = s_ref[0] * jnp.exp(m_old - m_new) + jnp.sum(jnp.exp(logits - m_new))
    m_ref[0] = m_new

    @pl.when(i == _NT - 1)
    def _():
        out_ref[...] = out_ref[...] - (m_ref[0] + jnp.log(s_ref[0]))


def _tc_full(idx_sref, table_ref, w1_ref, b1_ref, w2_ref, b2_ref, out_ref,
             rows_v, h_ref, m_ref, s_ref, sem):
    i = pl.program_id(0)

    @pl.when(i == 0)
    def _():
        for j in range(_CTX):
            pltpu.make_async_copy(
                table_ref.at[pl.ds(idx_sref[j], 1), :],
                rows_v.at[pl.ds(j, 1), :], sem).start()
        for j in range(_CTX):
            pltpu.make_async_copy(
                table_ref.at[pl.ds(idx_sref[j], 1), :],
                rows_v.at[pl.ds(j, 1), :], sem).wait()
        emb = jnp.sum(rows_v[...], axis=0, keepdims=True)
        h = lax.dot_general(emb, w1_ref[...], (((1,), (1,)), ((), ())),
                            preferred_element_type=jnp.float32)
        h_ref[...] = jnp.maximum(h + b1_ref[...], 0.0)
        m_ref[0] = -jnp.inf
        s_ref[0] = 0.0

    logits = lax.dot_general(h_ref[...], w2_ref[...], (((1,), (1,)), ((), ())),
                             preferred_element_type=jnp.float32) + b2_ref[0]
    out_ref[pl.ds(i, 1), :] = logits

    m_old = m_ref[0]
    m_new = jnp.maximum(m_old, jnp.max(logits))
    s_ref[0] = s_ref[0] * jnp.exp(m_old - m_new) + jnp.sum(jnp.exp(logits - m_new))
    m_ref[0] = m_new

    @pl.when(i == _NT - 1)
    def _():
        out_ref[...] = out_ref[...] - (m_ref[0] + jnp.log(s_ref[0]))


_VT2 = 10000
_NT2 = _VOCAB // (2 * _VT2)


def _tc_dual(idx_sref, table_ref, w1_ref, b1_ref, w2a_ref, w2b_ref,
             b2a_ref, b2b_ref, out_ref, rows_v, h_ref, m_ref, s_ref, sem):
    i = pl.program_id(0)

    @pl.when(i == 0)
    def _():
        for j in range(_CTX):
            pltpu.make_async_copy(
                table_ref.at[pl.ds(idx_sref[j], 1), :],
                rows_v.at[pl.ds(j, 1), :], sem).start()
        for j in range(_CTX):
            pltpu.make_async_copy(
                table_ref.at[pl.ds(idx_sref[j], 1), :],
                rows_v.at[pl.ds(j, 1), :], sem).wait()
        emb = jnp.sum(rows_v[...], axis=0, keepdims=True)
        h = lax.dot_general(emb, w1_ref[...], (((1,), (1,)), ((), ())),
                            preferred_element_type=jnp.float32)
        h_ref[...] = jnp.maximum(h + b1_ref[...], 0.0)
        m_ref[0] = -jnp.inf
        s_ref[0] = 0.0

    h = h_ref[...]
    la = lax.dot_general(h, w2a_ref[...], (((1,), (1,)), ((), ())),
                         preferred_element_type=jnp.float32) + b2a_ref[0]
    lb = lax.dot_general(h, w2b_ref[...], (((1,), (1,)), ((), ())),
                         preferred_element_type=jnp.float32) + b2b_ref[0]
    out_ref[pl.ds(2 * i, 1), :] = la
    out_ref[pl.ds(2 * i + 1, 1), :] = lb

    m_old = m_ref[0]
    m_new = jnp.maximum(m_old, jnp.maximum(jnp.max(la), jnp.max(lb)))
    s_ref[0] = (s_ref[0] * jnp.exp(m_old - m_new)
                + jnp.sum(jnp.exp(la - m_new)) + jnp.sum(jnp.exp(lb - m_new)))
    m_ref[0] = m_new

    @pl.when(i == _NT2 - 1)
    def _():
        out_ref[...] = out_ref[...] - (m_ref[0] + jnp.log(s_ref[0]))


def _tc_dual_call(idx, emb_table, W1, b1, W2, b2):
    b2r = b2.reshape(2 * _NT2, 1, _VT2)
    return pl.pallas_call(
        _tc_dual,
        grid_spec=pltpu.PrefetchScalarGridSpec(
            num_scalar_prefetch=1,
            grid=(_NT2,),
            in_specs=[
                pl.BlockSpec(memory_space=pltpu.MemorySpace.HBM),
                pl.BlockSpec((_HIDDEN, _EMBED), lambda i, s: (0, 0)),
                pl.BlockSpec((1, _HIDDEN), lambda i, s: (0, 0)),
                pl.BlockSpec((_VT2, _HIDDEN), lambda i, s: (2 * i, 0)),
                pl.BlockSpec((_VT2, _HIDDEN), lambda i, s: (2 * i + 1, 0)),
                pl.BlockSpec((1, 1, _VT2), lambda i, s: (2 * i, 0, 0)),
                pl.BlockSpec((1, 1, _VT2), lambda i, s: (2 * i + 1, 0, 0)),
            ],
            out_specs=pl.BlockSpec((2 * _NT2, _VT2), lambda i, s: (0, 0)),
            scratch_shapes=[
                pltpu.VMEM((_CTX, _EMBED), jnp.float32),
                pltpu.VMEM((1, _HIDDEN), jnp.float32),
                pltpu.SMEM((1,), jnp.float32),
                pltpu.SMEM((1,), jnp.float32),
                pltpu.SemaphoreType.DMA,
            ],
        ),
        out_shape=jax.ShapeDtypeStruct((2 * _NT2, _VT2), jnp.float32),
    )(idx, emb_table, W1, b1.reshape(1, _HIDDEN), W2, W2, b2r, b2r)


def _tc_full_call(idx, emb_table, W1, b1, W2, b2):
    return pl.pallas_call(
        _tc_full,
        grid_spec=pltpu.PrefetchScalarGridSpec(
            num_scalar_prefetch=1,
            grid=(_NT,),
            in_specs=[
                pl.BlockSpec(memory_space=pltpu.MemorySpace.HBM),
                pl.BlockSpec((_HIDDEN, _EMBED), lambda i, s: (0, 0)),
                pl.BlockSpec((1, _HIDDEN), lambda i, s: (0, 0)),
                pl.BlockSpec((_VT, _HIDDEN), lambda i, s: (i, 0)),
                pl.BlockSpec((1, 1, _VT), lambda i, s: (i, 0, 0)),
            ],
            out_specs=pl.BlockSpec((_NT, _VT), lambda i, s: (0, 0)),
            scratch_shapes=[
                pltpu.VMEM((_CTX, _EMBED), jnp.float32),
                pltpu.VMEM((1, _HIDDEN), jnp.float32),
                pltpu.SMEM((1,), jnp.float32),
                pltpu.SMEM((1,), jnp.float32),
                pltpu.SemaphoreType.DMA,
            ],
        ),
        out_shape=jax.ShapeDtypeStruct((_NT, _VT), jnp.float32),
    )(idx, emb_table, W1, b1.reshape(1, _HIDDEN), W2, b2.reshape(_NT, 1, _VT))


def _tc_call(partials, W1, b1, W2, b2):
    return pl.pallas_call(
        _tc_mlp,
        grid=(_NT,),
        in_specs=[
            pl.BlockSpec((_NW, _EMBED), lambda i: (0, 0)),
            pl.BlockSpec((_HIDDEN, _EMBED), lambda i: (0, 0)),
            pl.BlockSpec((1, _HIDDEN), lambda i: (0, 0)),
            pl.BlockSpec((_VT, _HIDDEN), lambda i: (i, 0)),
            pl.BlockSpec((1, 1, _VT), lambda i: (i, 0, 0)),
        ],
        out_specs=pl.BlockSpec((_NT, _VT), lambda i: (0, 0)),
        out_shape=jax.ShapeDtypeStruct((_NT, _VT), jnp.float32),
        scratch_shapes=[
            pltpu.VMEM((1, _HIDDEN), jnp.float32),
            pltpu.SMEM((1,), jnp.float32),
            pltpu.SMEM((1,), jnp.float32),
        ],
    )(partials, W1, b1.reshape(1, _HIDDEN), W2, b2.reshape(_NT, 1, _VT))


def kernel(inputs, emb_table, W1, b1, W2, b2):
    idx = inputs.astype(jnp.int32)
    out2d = _tc_full_call(idx, emb_table, W1, b1, W2, b2)
    return out2d.reshape(1, _VOCAB)


# speedup vs baseline: 1.0395x; 1.0395x over previous
import jax
import jax.numpy as jnp
from jax import lax
from jax.experimental import pallas as pl
from jax.experimental.pallas import tpu as pltpu

_VOCAB = 100000
_EMBED = 128
_HIDDEN = 128
_CTX = 200

_VT = 20000
_NT = _VOCAB // _VT


def _cbow_kernel(idx_sref, table_ref, w1_ref, b1_ref, w2_ref, b2_ref, out_ref,
                 rows_v, h_ref, m_ref, s_ref, sem):
    i = pl.program_id(0)

    @pl.when(i == 0)
    def _():
        for j in range(_CTX):
            pltpu.make_async_copy(
                table_ref.at[pl.ds(idx_sref[j], 1), :],
                rows_v.at[pl.ds(j, 1), :], sem).start()
        for j in range(_CTX):
            pltpu.make_async_copy(
                table_ref.at[pl.ds(idx_sref[j], 1), :],
                rows_v.at[pl.ds(j, 1), :], sem).wait()
        emb = jnp.sum(rows_v[...], axis=0, keepdims=True)
        h = lax.dot_general(emb, w1_ref[...], (((1,), (1,)), ((), ())),
                            preferred_element_type=jnp.float32)
        h_ref[...] = jnp.maximum(h + b1_ref[...], 0.0)
        m_ref[0] = -jnp.inf
        s_ref[0] = 0.0

    logits = lax.dot_general(h_ref[...], w2_ref[...], (((1,), (1,)), ((), ())),
                             preferred_element_type=jnp.float32) + b2_ref[0]
    out_ref[pl.ds(i, 1), :] = logits

    m_old = m_ref[0]
    m_new = jnp.maximum(m_old, jnp.max(logits))
    s_ref[0] = s_ref[0] * jnp.exp(m_old - m_new) + jnp.sum(jnp.exp(logits - m_new))
    m_ref[0] = m_new

    @pl.when(i == _NT - 1)
    def _():
        out_ref[...] = out_ref[...] - (m_ref[0] + jnp.log(s_ref[0]))


def kernel(inputs, emb_table, W1, b1, W2, b2):
    idx = inputs.astype(jnp.int32)
    out2d = pl.pallas_call(
        _cbow_kernel,
        grid_spec=pltpu.PrefetchScalarGridSpec(
            num_scalar_prefetch=1,
            grid=(_NT,),
            in_specs=[
                pl.BlockSpec(memory_space=pltpu.MemorySpace.HBM),
                pl.BlockSpec((_HIDDEN, _EMBED), lambda i, s: (0, 0)),
                pl.BlockSpec((1, _HIDDEN), lambda i, s: (0, 0)),
                pl.BlockSpec((_VT, _HIDDEN), lambda i, s: (i, 0)),
                pl.BlockSpec((1, 1, _VT), lambda i, s: (i, 0, 0)),
            ],
            out_specs=pl.BlockSpec((_NT, _VT), lambda i, s: (0, 0)),
            scratch_shapes=[
                pltpu.VMEM((_CTX, _EMBED), jnp.float32),
                pltpu.VMEM((1, _HIDDEN), jnp.float32),
                pltpu.SMEM((1,), jnp.float32),
                pltpu.SMEM((1,), jnp.float32),
                pltpu.SemaphoreType.DMA,
            ],
        ),
        out_shape=jax.ShapeDtypeStruct((_NT, _VT), jnp.float32),
    )(idx, emb_table, W1, b1.reshape(1, _HIDDEN), W2, b2.reshape(_NT, 1, _VT))
    return out2d.reshape(1, _VOCAB)
